# baseline (speedup 1.0000x reference)
import jax
import jax.numpy as jnp
from jax.experimental import pallas as pl
from jax.experimental.pallas import tpu as pltpu

_N = 16384
_F = 1024
_BR = 512


def _body(ptr_ref, state_ref, mem_ref, out_ref, ctx_ref, acc_ref):
    i = pl.program_id(0)
    block = mem_ref[...]
    out_ref[...] = block

    @pl.when(i == 0)
    def _init():
        acc_ref[...] = jnp.zeros_like(acc_ref)

    acc_ref[...] += jnp.sum(block, axis=0, keepdims=True)

    ptr = ptr_ref[0]

    @pl.when(i == ptr // _BR)
    def _scatter():
        local = ptr % _BR
        state = state_ref[...]
        acc_ref[...] += state - mem_ref[pl.ds(local, 1), :]
        out_ref[pl.ds(local, 1), :] = state

    @pl.when(i == pl.num_programs(0) - 1)
    def _emit():
        ctx_ref[...] = acc_ref[...] * (1.0 / _N)


def kernel(new_state, memory_buffer, memory_ptr):
    ptr = jnp.asarray(memory_ptr, jnp.int32).reshape((1,))
    grid_spec = pltpu.PrefetchScalarGridSpec(
        num_scalar_prefetch=1,
        grid=(_N // _BR,),
        in_specs=[
            pl.BlockSpec((1, _F), lambda i, p: (0, 0)),
            pl.BlockSpec((_BR, _F), lambda i, p: (i, 0)),
        ],
        out_specs=[
            pl.BlockSpec((_BR, _F), lambda i, p: (i, 0)),
            pl.BlockSpec((1, _F), lambda i, p: (0, 0)),
        ],
        scratch_shapes=[pltpu.VMEM((1, _F), jnp.float32)],
    )
    mem_out, ctx = pl.pallas_call(
        _body,
        grid_spec=grid_spec,
        out_shape=[
            jax.ShapeDtypeStruct((_N, _F), jnp.float32),
            jax.ShapeDtypeStruct((1, _F), jnp.float32),
        ],
    )(ptr, new_state, memory_buffer)
    new_ptr = (memory_ptr + 1) % _N
    return (ctx.reshape(_F), mem_out, new_ptr)


# speedup vs baseline: 1.1150x; 1.1150x over previous
import jax
import jax.numpy as jnp
from jax.experimental import pallas as pl
from jax.experimental.pallas import tpu as pltpu

_N = 16384
_F = 1024
_BR = 1024


def _body(ptr_ref, state_ref, mem_ref, out_ref, ctx_ref, acc_ref):
    i = pl.program_id(0)
    block = mem_ref[...]
    out_ref[...] = block

    @pl.when(i == 0)
    def _init():
        acc_ref[...] = jnp.zeros_like(acc_ref)

    acc_ref[...] += jnp.sum(block, axis=0, keepdims=True)

    ptr = ptr_ref[0]

    @pl.when(i == ptr // _BR)
    def _scatter():
        local = ptr % _BR
        state = state_ref[...]
        acc_ref[...] += state - mem_ref[pl.ds(local, 1), :]
        out_ref[pl.ds(local, 1), :] = state

    @pl.when(i == pl.num_programs(0) - 1)
    def _emit():
        ctx_ref[...] = acc_ref[...] * (1.0 / _N)


def kernel(new_state, memory_buffer, memory_ptr):
    ptr = jnp.asarray(memory_ptr, jnp.int32).reshape((1,))
    grid_spec = pltpu.PrefetchScalarGridSpec(
        num_scalar_prefetch=1,
        grid=(_N // _BR,),
        in_specs=[
            pl.BlockSpec((1, _F), lambda i, p: (0, 0)),
            pl.BlockSpec((_BR, _F), lambda i, p: (i, 0)),
        ],
        out_specs=[
            pl.BlockSpec((_BR, _F), lambda i, p: (i, 0)),
            pl.BlockSpec((1, _F), lambda i, p: (0, 0)),
        ],
        scratch_shapes=[pltpu.VMEM((1, _F), jnp.float32)],
    )
    mem_out, ctx = pl.pallas_call(
        _body,
        grid_spec=grid_spec,
        out_shape=[
            jax.ShapeDtypeStruct((_N, _F), jnp.float32),
            jax.ShapeDtypeStruct((1, _F), jnp.float32),
        ],
    )(ptr, new_state, memory_buffer)
    new_ptr = (memory_ptr + 1) % _N
    return (ctx.reshape(_F), mem_out, new_ptr)


# speedup vs baseline: 1.1405x; 1.0228x over previous
import jax
import jax.numpy as jnp
from jax.experimental import pallas as pl
from jax.experimental.pallas import tpu as pltpu

_N = 16384
_F = 1024
_BR = 2048


def _body(ptr_ref, state_ref, mem_ref, out_ref, ctx_ref, acc_ref):
    i = pl.program_id(0)
    block = mem_ref[...]
    out_ref[...] = block

    @pl.when(i == 0)
    def _init():
        acc_ref[...] = jnp.zeros_like(acc_ref)

    acc_ref[...] += jnp.sum(block, axis=0, keepdims=True)

    ptr = ptr_ref[0]

    @pl.when(i == ptr // _BR)
    def _scatter():
        local = ptr % _BR
        state = state_ref[...]
        acc_ref[...] += state - mem_ref[pl.ds(local, 1), :]
        out_ref[pl.ds(local, 1), :] = state

    @pl.when(i == pl.num_programs(0) - 1)
    def _emit():
        ctx_ref[...] = acc_ref[...] * (1.0 / _N)


def kernel(new_state, memory_buffer, memory_ptr):
    ptr = jnp.asarray(memory_ptr, jnp.int32).reshape((1,))
    grid_spec = pltpu.PrefetchScalarGridSpec(
        num_scalar_prefetch=1,
        grid=(_N // _BR,),
        in_specs=[
            pl.BlockSpec((1, _F), lambda i, p: (0, 0)),
            pl.BlockSpec((_BR, _F), lambda i, p: (i, 0)),
        ],
        out_specs=[
            pl.BlockSpec((_BR, _F), lambda i, p: (i, 0)),
            pl.BlockSpec((1, _F), lambda i, p: (0, 0)),
        ],
        scratch_shapes=[pltpu.VMEM((1, _F), jnp.float32)],
    )
    mem_out, ctx = pl.pallas_call(
        _body,
        grid_spec=grid_spec,
        out_shape=[
            jax.ShapeDtypeStruct((_N, _F), jnp.float32),
            jax.ShapeDtypeStruct((1, _F), jnp.float32),
        ],
    )(ptr, new_state, memory_buffer)
    new_ptr = (memory_ptr + 1) % _N
    return (ctx.reshape(_F), mem_out, new_ptr)
